# baseline (speedup 1.0000x reference)
import dataclasses
import functools

import jax
import jax.numpy as jnp
from jax.experimental import pallas as pl
from jax.experimental.pallas import tpu as pltpu
from jax.experimental.pallas import tpu_sc as plsc

_GW = 96
_NBUF = 2
_NW = 32
_KWIN = 2000


def _sc_invert_and_gather(x_pad, kin_flat, kout_flat, n, n_pad, k):
    m_pad = (k - 1) * n_pad
    ipt = m_pad // _NW
    nwin = ipt // _GW
    mesh = plsc.VectorSubcoreMesh(core_axis_name="c", subcore_axis_name="s")

    row_t = pltpu.VMEM((_GW, 128), x_pad.dtype)
    scratch = [
        pltpu.VMEM((n_pad,), jnp.int32),
        pltpu.VMEM((_KWIN,), jnp.int32),
        pltpu.VMEM((_KWIN,), jnp.int32),
        pltpu.VMEM((ipt,), jnp.int32),
    ] + [row_t] * _NBUF + [pltpu.SemaphoreType.DMA] * (2 * _NBUF)

    out_types = (
        jax.ShapeDtypeStruct((m_pad, 128), x_pad.dtype),
        jax.ShapeDtypeStruct((m_pad,), jnp.int32),
    )

    cp = pltpu.CompilerParams()
    if "needs_layout_passes" in pltpu.CompilerParams.__dataclass_fields__:
        cp = dataclasses.replace(cp, needs_layout_passes=False)

    @functools.partial(pl.kernel, out_type=out_types, mesh=mesh,
                       scratch_types=scratch, compiler_params=cp)
    def sc_kernel(x_hbm, kin_hbm, kout_hbm, g_hbm, inv_hbm,
                  a_k, kin_v, kout_v, idx_v, r0, r1, g0, g1, w0, w1):
        rows = (r0, r1)
        gsem = (g0, g1)
        wsem = (w0, w1)
        cid = jax.lax.axis_index("c")
        sid = jax.lax.axis_index("s")
        wid = sid * 2 + cid
        base = wid * ipt

        for rep in range(2):
            kk = rep * 16 + sid
            rk = kk + jnp.where(kk >= k // 2, 1, 0).astype(kk.dtype)

            @pl.when(kk < k - 1)
            def _():
                @pl.loop(0, n_pad, step=16)
                def _(i):
                    a_k[pl.ds(i, 16)] = jnp.full((16,), n, jnp.int32)

                @pl.loop(0, n, step=_KWIN)
                def _(w):
                    pltpu.sync_copy(
                        kin_hbm.at[pl.ds(rk * n + w, _KWIN)], kin_v)
                    pltpu.sync_copy(
                        kout_hbm.at[pl.ds(rk * n + w, _KWIN)], kout_v)

                    @pl.loop(0, _KWIN, step=16)
                    def _(q):
                        plsc.store_scatter(
                            a_k, [kout_v[pl.ds(q, 16)]], kin_v[pl.ds(q, 16)]
                        )

                pltpu.sync_copy(a_k, inv_hbm.at[pl.ds(kk * n_pad, n_pad)])

        plsc.subcore_barrier()

        pltpu.sync_copy(inv_hbm.at[pl.ds(base, ipt)], idx_v)

        def o_slice(j):
            return g_hbm.at[pl.ds(base + j * _GW, _GW)]

        def fire_gather(j, b):
            pltpu.make_async_copy(
                x_hbm.at[idx_v.at[pl.ds(j * _GW, _GW)]], rows[b], gsem[b]
            ).start()

        def writeback(j, b):
            pltpu.make_async_copy(
                x_hbm.at[idx_v.at[pl.ds(j * _GW, _GW)]], rows[b], gsem[b]
            ).wait()
            pltpu.make_async_copy(rows[b], o_slice(j), wsem[b]).start()

        def wait_wb(j, b):
            pltpu.make_async_copy(rows[b], o_slice(j), wsem[b]).wait()

        @pl.loop(0, nwin - 1, step=_NBUF)
        def _(j0):
            for b in range(_NBUF):
                j = j0 + b

                @pl.when(j >= _NBUF)
                def _():
                    wait_wb(j - _NBUF, b)

                fire_gather(j, b)

                @pl.when(j >= _NBUF - 1)
                def _():
                    writeback(j - (_NBUF - 1), (b + 1) % _NBUF)

        last = nwin - 1
        lb = last % _NBUF
        wait_wb(last - _NBUF, lb)
        fire_gather(last, lb)
        for j in range(last - (_NBUF - 1), last + 1):
            writeback(j, j % _NBUF)
        for j in range(last - (_NBUF - 1), last + 1):
            wait_wb(j, j % _NBUF)

    return sc_kernel(x_pad, kin_flat, kout_flat)[0]


def _tc_matmul(g3, x_tc, w_r, bias2d, n_pad, k, bi):
    def body(g_ref, x_ref, w_ref, b_ref, o_ref):
        acc = jnp.dot(x_ref[...], w_ref[0], preferred_element_type=jnp.float32)
        for p in range(k - 1):
            acc = acc + jnp.dot(
                g_ref[p], w_ref[p + 1], preferred_element_type=jnp.float32
            )
        o_ref[...] = acc + b_ref[...]

    return pl.pallas_call(
        body,
        grid=(n_pad // bi,),
        in_specs=[
            pl.BlockSpec((k - 1, bi, 128), lambda i: (0, i, 0)),
            pl.BlockSpec((bi, 128), lambda i: (i, 0)),
            pl.BlockSpec((k, 128, 128), lambda i: (0, 0, 0)),
            pl.BlockSpec((1, 128), lambda i: (0, 0)),
        ],
        out_specs=pl.BlockSpec((bi, 128), lambda i: (i, 0)),
        out_shape=jax.ShapeDtypeStruct((n_pad, 128), jnp.float32),
    )(g3, x_tc, w_r, bias2d)


def kernel(x, weight, bias, kmap_in, kmap_out):
    n, cin = x.shape
    k, _, cout = weight.shape

    n_pad = ((n + 1535) // 1536) * 1536

    x_tc = jnp.concatenate(
        [x, jnp.zeros((n_pad - n, cin), x.dtype)], axis=0
    )

    g = _sc_invert_and_gather(
        x_tc,
        kmap_in.astype(jnp.int32).reshape(-1),
        kmap_out.astype(jnp.int32).reshape(-1),
        n, n_pad, k,
    )
    g3 = g.reshape(k - 1, n_pad, cin)

    c = k // 2
    w_r = jnp.concatenate([weight[c:c + 1], weight[:c], weight[c + 1:]])

    out_full = _tc_matmul(
        g3, x_tc, w_r, bias.reshape(1, cout), n_pad, k, bi=512,
    )
    return out_full[:n]


# speedup vs baseline: 1.0155x; 1.0155x over previous
import dataclasses
import functools

import jax
import jax.numpy as jnp
from jax.experimental import pallas as pl
from jax.experimental.pallas import tpu as pltpu
from jax.experimental.pallas import tpu_sc as plsc

_GW = 96
_NBUF = 2
_NW = 32
_KWIN = 2000


def _sc_invert_and_gather(x_pad, kin_flat, kout_flat, n, n_pad, k):
    m_pad = k * n_pad
    ipt = m_pad // _NW
    nwin = ipt // _GW
    mesh = plsc.VectorSubcoreMesh(core_axis_name="c", subcore_axis_name="s")

    row_t = pltpu.VMEM((_GW, 128), x_pad.dtype)
    scratch = [
        pltpu.VMEM((n_pad,), jnp.int32),
        pltpu.VMEM((_KWIN,), jnp.int32),
        pltpu.VMEM((_KWIN,), jnp.int32),
        pltpu.VMEM((ipt,), jnp.int32),
    ] + [row_t] * _NBUF + [pltpu.SemaphoreType.DMA] * (2 * _NBUF)

    out_types = (
        jax.ShapeDtypeStruct((m_pad, 128), x_pad.dtype),
        jax.ShapeDtypeStruct((m_pad,), jnp.int32),
    )

    cp = pltpu.CompilerParams()
    if "needs_layout_passes" in pltpu.CompilerParams.__dataclass_fields__:
        cp = dataclasses.replace(cp, needs_layout_passes=False)

    @functools.partial(pl.kernel, out_type=out_types, mesh=mesh,
                       scratch_types=scratch, compiler_params=cp)
    def sc_kernel(x_hbm, kin_hbm, kout_hbm, g_hbm, inv_hbm,
                  a_k, kin_v, kout_v, idx_v, r0, r1, g0, g1, w0, w1):
        rows = (r0, r1)
        gsem = (g0, g1)
        wsem = (w0, w1)
        cid = jax.lax.axis_index("c")
        sid = jax.lax.axis_index("s")
        wid = sid * 2 + cid
        base = wid * ipt

        for rep in range(2):
            kk = rep * 16 + sid

            @pl.when(kk < k)
            def _():
                @pl.loop(0, n_pad, step=16)
                def _(i):
                    a_k[pl.ds(i, 16)] = jnp.full((16,), n, jnp.int32)

                @pl.loop(0, n, step=_KWIN)
                def _(w):
                    pltpu.sync_copy(
                        kin_hbm.at[pl.ds(kk * n + w, _KWIN)], kin_v)
                    pltpu.sync_copy(
                        kout_hbm.at[pl.ds(kk * n + w, _KWIN)], kout_v)

                    @pl.loop(0, _KWIN, step=16)
                    def _(q):
                        plsc.store_scatter(
                            a_k, [kout_v[pl.ds(q, 16)]], kin_v[pl.ds(q, 16)]
                        )

                pltpu.sync_copy(a_k, inv_hbm.at[pl.ds(kk * n_pad, n_pad)])

        plsc.subcore_barrier()

        pltpu.sync_copy(inv_hbm.at[pl.ds(base, ipt)], idx_v)

        def o_slice(j):
            return g_hbm.at[pl.ds(base + j * _GW, _GW)]

        def fire_gather(j, b):
            pltpu.make_async_copy(
                x_hbm.at[idx_v.at[pl.ds(j * _GW, _GW)]], rows[b], gsem[b]
            ).start()

        def writeback(j, b):
            pltpu.make_async_copy(
                x_hbm.at[idx_v.at[pl.ds(j * _GW, _GW)]], rows[b], gsem[b]
            ).wait()
            pltpu.make_async_copy(rows[b], o_slice(j), wsem[b]).start()

        def wait_wb(j, b):
            pltpu.make_async_copy(rows[b], o_slice(j), wsem[b]).wait()

        @pl.loop(0, nwin - 1, step=_NBUF)
        def _(j0):
            for b in range(_NBUF):
                j = j0 + b

                @pl.when(j >= _NBUF)
                def _():
                    wait_wb(j - _NBUF, b)

                fire_gather(j, b)

                @pl.when(j >= _NBUF - 1)
                def _():
                    writeback(j - (_NBUF - 1), (b + 1) % _NBUF)

        last = nwin - 1
        lb = last % _NBUF
        wait_wb(last - _NBUF, lb)
        fire_gather(last, lb)
        for j in range(last - (_NBUF - 1), last + 1):
            writeback(j, j % _NBUF)
        for j in range(last - (_NBUF - 1), last + 1):
            wait_wb(j, j % _NBUF)

    return sc_kernel(x_pad, kin_flat, kout_flat)[0]


def _tc_matmul(g3, w, bias2d, n_pad, k, bi):
    def body(g_ref, w_ref, b_ref, o_ref):
        acc = jnp.dot(g_ref[0], w_ref[0], preferred_element_type=jnp.float32)
        for kk in range(1, k):
            acc = acc + jnp.dot(
                g_ref[kk], w_ref[kk], preferred_element_type=jnp.float32
            )
        o_ref[...] = acc + b_ref[...]

    return pl.pallas_call(
        body,
        grid=(n_pad // bi,),
        in_specs=[
            pl.BlockSpec((k, bi, 128), lambda i: (0, i, 0)),
            pl.BlockSpec((k, 128, 128), lambda i: (0, 0, 0)),
            pl.BlockSpec((1, 128), lambda i: (0, 0)),
        ],
        out_specs=pl.BlockSpec((bi, 128), lambda i: (i, 0)),
        out_shape=jax.ShapeDtypeStruct((n_pad, 128), jnp.float32),
    )(g3, w, bias2d)


def kernel(x, weight, bias, kmap_in, kmap_out):
    n, cin = x.shape
    k, _, cout = weight.shape

    n_pad = ((n + 1023) // 1024) * 1024

    x_pad = jnp.concatenate([x, jnp.zeros((1, cin), x.dtype)], axis=0)

    g = _sc_invert_and_gather(
        x_pad,
        kmap_in.astype(jnp.int32).reshape(-1),
        kmap_out.astype(jnp.int32).reshape(-1),
        n, n_pad, k,
    )
    g3 = g.reshape(k, n_pad, cin)

    out_full = _tc_matmul(
        g3, weight, bias.reshape(1, cout), n_pad, k, bi=448,
    )
    return out_full[:n]
